# baseline (speedup 1.0000x reference)
import jax
import jax.numpy as jnp
from jax.experimental import pallas as pl
from jax.experimental.pallas import tpu as pltpu
from jax.experimental.pallas import tpu_sc as plsc

N_OUT = 2048
K_IN = 8192
N_DELAY = 16

MASK_BO = 64
MM_BM = 256
MM_BN = 1024


def _roll_left(x, s):
    return pltpu.roll(x, x.shape[1] - s, 1)


def _mask_kernel(w_ref, o_ref):
    w = w_ref[...]
    a = jnp.abs(w)
    lane_mod = (jax.lax.broadcasted_iota(jnp.int32, (1, K_IN), 1)
                & (N_DELAY - 1))
    neg = jnp.full_like(a, -1.0)
    ok = lane_mod < N_DELAY - 1
    b1 = _roll_left(a, 1)
    t1 = jnp.where(ok, jnp.maximum(a, b1), a)
    t2 = jnp.where(ok, jnp.minimum(a, b1), neg)
    ok = lane_mod < N_DELAY - 2
    b1 = _roll_left(t1, 2)
    b2 = _roll_left(t2, 2)
    m3 = jnp.maximum(jnp.minimum(t1, b2), jnp.minimum(t2, b1))
    m2 = jnp.maximum(jnp.minimum(t1, b1), jnp.maximum(t2, b2))
    t3 = jnp.where(ok, m3, neg)
    m1 = jnp.maximum(t1, b1)
    t2 = jnp.where(ok, m2, t2)
    t1 = jnp.where(ok, m1, t1)
    for s in (4, 8):
        ok = lane_mod < N_DELAY - s
        b1 = _roll_left(t1, s)
        b2 = _roll_left(t2, s)
        b3 = _roll_left(t3, s)
        m3 = jnp.maximum(jnp.maximum(t3, b3),
                         jnp.maximum(jnp.minimum(t1, b2), jnp.minimum(t2, b1)))
        m2 = jnp.maximum(jnp.minimum(t1, b1), jnp.maximum(t2, b2))
        m1 = jnp.maximum(t1, b1)
        t1 = jnp.where(ok, m1, t1)
        t2 = jnp.where(ok, m2, t2)
        t3 = jnp.where(ok, m3, t3)
    for s in (1, 2, 4, 8):
        prv = pltpu.roll(t3, s, 1)
        t3 = jnp.where(lane_mod >= s, prv, t3)
    o_ref[...] = jnp.where(a >= t3, w, 0.0).astype(jnp.bfloat16)


def _matmul_kernel(x_ref, w_ref, o_ref):
    xb = x_ref[...].astype(jnp.bfloat16)
    o_ref[...] = jax.lax.dot_general(
        xb, w_ref[...],
        dimension_numbers=(((1,), (1,)), ((), ())),
        preferred_element_type=jnp.float32)


SC_BR = 8
SC_BC = 512


def _sc_mask(W):
    mesh = plsc.VectorSubcoreMesh(core_axis_name="c", subcore_axis_name="s")

    @pl.kernel(out_type=jax.ShapeDtypeStruct((N_OUT, K_IN), jnp.float32),
               mesh=mesh,
               compiler_params=pltpu.CompilerParams(needs_layout_passes=False))
    def sc_mask_kernel(w_hbm, o_hbm):
        def body(w_vmem, o_vmem):
            @pl.loop(0, SC_BR)
            def _row(r):
                @pl.loop(0, SC_BC, step=N_DELAY)
                def _grp(c):
                    w = w_vmem[r, pl.ds(c, N_DELAY)]
                    a = jnp.abs(w)
                    srt = plsc.sort_key_val(a, a, descending=True)[0]
                    thr = srt[2]
                    o_vmem[r, pl.ds(c, N_DELAY)] = jnp.where(a >= thr, w, 0.0)

        pltpu.emit_pipeline(
            body,
            grid=(N_OUT // SC_BR, K_IN // SC_BC),
            in_specs=[pl.BlockSpec((SC_BR, SC_BC), lambda i, j: (i, j))],
            out_specs=[pl.BlockSpec((SC_BR, SC_BC), lambda i, j: (i, j))],
            core_axis_name=("c", "s"),
            dimension_semantics=(pltpu.PARALLEL, pltpu.PARALLEL),
        )(w_hbm, o_hbm)

    return sc_mask_kernel(W)


def _cast_kernel(w_ref, o_ref):
    o_ref[...] = w_ref[...].astype(jnp.bfloat16)


def kernel(x, W):
    M = x.shape[0]
    Wm32 = _sc_mask(W)
    Wm = pl.pallas_call(
        _cast_kernel,
        grid=(N_OUT // 256,),
        in_specs=[pl.BlockSpec((256, K_IN), lambda i: (i, 0))],
        out_specs=pl.BlockSpec((256, K_IN), lambda i: (i, 0)),
        out_shape=jax.ShapeDtypeStruct((N_OUT, K_IN), jnp.bfloat16),
    )(Wm32)
    out = pl.pallas_call(
        _matmul_kernel,
        grid=(N_OUT // MM_BN, M // MM_BM),
        in_specs=[
            pl.BlockSpec((MM_BM, K_IN), lambda n, m: (m, 0)),
            pl.BlockSpec((MM_BN, K_IN), lambda n, m: (n, 0)),
        ],
        out_specs=pl.BlockSpec((MM_BM, MM_BN), lambda n, m: (m, n)),
        out_shape=jax.ShapeDtypeStruct((M, N_OUT), jnp.float32),
    )(x, Wm)
    return out


# speedup vs baseline: 1.5919x; 1.5919x over previous
import jax
import jax.numpy as jnp
from jax.experimental import pallas as pl
from jax.experimental.pallas import tpu as pltpu
from jax.experimental.pallas import tpu_sc as plsc

N_OUT = 2048
K_IN = 8192
N_DELAY = 16

TC_ROWS = 1536
SC_ROWS = N_OUT - TC_ROWS
MASK_BO = 64
MM_BM = 256
MM_BN = 512

SC_BR = 8
SC_BC = 512


def _roll_left(x, s):
    return pltpu.roll(x, x.shape[1] - s, 1)


def _mask_kernel(w_ref, o_ref):
    w = w_ref[...]
    a = jnp.abs(w)
    lane_mod = (jax.lax.broadcasted_iota(jnp.int32, (1, K_IN), 1)
                & (N_DELAY - 1))
    neg = jnp.full_like(a, -1.0)
    ok = lane_mod < N_DELAY - 1
    b1 = _roll_left(a, 1)
    t1 = jnp.where(ok, jnp.maximum(a, b1), a)
    t2 = jnp.where(ok, jnp.minimum(a, b1), neg)
    ok = lane_mod < N_DELAY - 2
    b1 = _roll_left(t1, 2)
    b2 = _roll_left(t2, 2)
    m3 = jnp.maximum(jnp.minimum(t1, b2), jnp.minimum(t2, b1))
    m2 = jnp.maximum(jnp.minimum(t1, b1), jnp.maximum(t2, b2))
    t3 = jnp.where(ok, m3, neg)
    m1 = jnp.maximum(t1, b1)
    t2 = jnp.where(ok, m2, t2)
    t1 = jnp.where(ok, m1, t1)
    for s in (4, 8):
        ok = lane_mod < N_DELAY - s
        b1 = _roll_left(t1, s)
        b2 = _roll_left(t2, s)
        b3 = _roll_left(t3, s)
        m3 = jnp.maximum(jnp.maximum(t3, b3),
                         jnp.maximum(jnp.minimum(t1, b2), jnp.minimum(t2, b1)))
        m2 = jnp.maximum(jnp.minimum(t1, b1), jnp.maximum(t2, b2))
        m1 = jnp.maximum(t1, b1)
        t1 = jnp.where(ok, m1, t1)
        t2 = jnp.where(ok, m2, t2)
        t3 = jnp.where(ok, m3, t3)
    for s in (1, 2, 4, 8):
        prv = pltpu.roll(t3, s, 1)
        t3 = jnp.where(lane_mod >= s, prv, t3)
    o_ref[...] = jnp.where(a >= t3, w, 0.0).astype(jnp.bfloat16)


def _sc_mask(W):
    mesh = plsc.VectorSubcoreMesh(core_axis_name="c", subcore_axis_name="s")
    row0 = TC_ROWS // SC_BR

    @pl.kernel(out_type=jax.ShapeDtypeStruct((SC_ROWS, K_IN), jnp.float32),
               mesh=mesh,
               compiler_params=pltpu.CompilerParams(needs_layout_passes=False))
    def sc_mask_kernel(w_hbm, o_hbm):
        def body(w_vmem, o_vmem):
            @pl.loop(0, SC_BR)
            def _row(r):
                @pl.loop(0, SC_BC, step=N_DELAY)
                def _grp(c):
                    w = w_vmem[r, pl.ds(c, N_DELAY)]
                    a = jnp.abs(w)
                    srt = plsc.sort_key_val(a, a, descending=True)[0]
                    thr = srt[2]
                    o_vmem[r, pl.ds(c, N_DELAY)] = jnp.where(a >= thr, w, 0.0)

        pltpu.emit_pipeline(
            body,
            grid=(SC_ROWS // SC_BR, K_IN // SC_BC),
            in_specs=[pl.BlockSpec((SC_BR, SC_BC),
                                   lambda i, j: (i + row0, j))],
            out_specs=[pl.BlockSpec((SC_BR, SC_BC), lambda i, j: (i, j))],
            core_axis_name=("c", "s"),
            dimension_semantics=(pltpu.PARALLEL, pltpu.PARALLEL),
        )(w_hbm, o_hbm)

    return sc_mask_kernel(W)


def _cast_kernel(w_ref, o_ref):
    o_ref[...] = w_ref[...].astype(jnp.bfloat16)


def _matmul_kernel(x_ref, w_ref, o_ref):
    xb = x_ref[...].astype(jnp.bfloat16)
    o_ref[...] = jax.lax.dot_general(
        xb, w_ref[...],
        dimension_numbers=(((1,), (1,)), ((), ())),
        preferred_element_type=jnp.float32)


def _matmul(x, Wm, bn):
    M, n_out = x.shape[0], Wm.shape[0]
    return pl.pallas_call(
        _matmul_kernel,
        grid=(n_out // bn, M // MM_BM),
        in_specs=[
            pl.BlockSpec((MM_BM, K_IN), lambda n, m: (m, 0)),
            pl.BlockSpec((bn, K_IN), lambda n, m: (n, 0)),
        ],
        out_specs=pl.BlockSpec((MM_BM, bn), lambda n, m: (m, n)),
        out_shape=jax.ShapeDtypeStruct((M, n_out), jnp.float32),
    )(x, Wm)


def kernel(x, W):
    wm_sc32 = _sc_mask(W)
    wm_tc = pl.pallas_call(
        _mask_kernel,
        grid=(TC_ROWS // MASK_BO,),
        in_specs=[pl.BlockSpec((MASK_BO, K_IN), lambda i: (i, 0))],
        out_specs=pl.BlockSpec((MASK_BO, K_IN), lambda i: (i, 0)),
        out_shape=jax.ShapeDtypeStruct((TC_ROWS, K_IN), jnp.bfloat16),
    )(W)
    wm_sc = pl.pallas_call(
        _cast_kernel,
        grid=(SC_ROWS // 256,),
        in_specs=[pl.BlockSpec((256, K_IN), lambda i: (i, 0))],
        out_specs=pl.BlockSpec((256, K_IN), lambda i: (i, 0)),
        out_shape=jax.ShapeDtypeStruct((SC_ROWS, K_IN), jnp.bfloat16),
    )(wm_sc32)
    y_tc = _matmul(x, wm_tc, MM_BN)
    y_sc = _matmul(x, wm_sc, MM_BN)
    return jnp.concatenate([y_tc, y_sc], axis=1)


# speedup vs baseline: 1.7583x; 1.1046x over previous
import jax
import jax.numpy as jnp
from jax.experimental import pallas as pl
from jax.experimental.pallas import tpu as pltpu

N_OUT = 2048
K_IN = 8192
N_DELAY = 16

MASK_BO = 64
MM_BM = 256
MM_BN = 2048


def _roll_left(x, s):
    return pltpu.roll(x, x.shape[1] - s, 1)


def _mask_kernel(w_ref, o_ref):
    w = w_ref[...]
    a = jnp.abs(w)
    lane_mod = (jax.lax.broadcasted_iota(jnp.int32, (1, K_IN), 1)
                & (N_DELAY - 1))
    neg = jnp.full_like(a, -1.0)
    ok = lane_mod < N_DELAY - 1
    b1 = _roll_left(a, 1)
    t1 = jnp.where(ok, jnp.maximum(a, b1), a)
    t2 = jnp.where(ok, jnp.minimum(a, b1), neg)
    ok = lane_mod < N_DELAY - 2
    b1 = _roll_left(t1, 2)
    b2 = _roll_left(t2, 2)
    m3 = jnp.maximum(jnp.minimum(t1, b2), jnp.minimum(t2, b1))
    m2 = jnp.maximum(jnp.minimum(t1, b1), jnp.maximum(t2, b2))
    t3 = jnp.where(ok, m3, neg)
    m1 = jnp.maximum(t1, b1)
    t2 = jnp.where(ok, m2, t2)
    t1 = jnp.where(ok, m1, t1)
    for s in (4, 8):
        ok = lane_mod < N_DELAY - s
        b1 = _roll_left(t1, s)
        b2 = _roll_left(t2, s)
        b3 = _roll_left(t3, s)
        m3 = jnp.maximum(jnp.maximum(t3, b3),
                         jnp.maximum(jnp.minimum(t1, b2), jnp.minimum(t2, b1)))
        m2 = jnp.maximum(jnp.minimum(t1, b1), jnp.maximum(t2, b2))
        m1 = jnp.maximum(t1, b1)
        t1 = jnp.where(ok, m1, t1)
        t2 = jnp.where(ok, m2, t2)
        t3 = jnp.where(ok, m3, t3)
    for s in (1, 2, 4, 8):
        prv = pltpu.roll(t3, s, 1)
        t3 = jnp.where(lane_mod >= s, prv, t3)
    o_ref[...] = jnp.where(a >= t3, w, 0.0).astype(jnp.bfloat16)


def _matmul_kernel(x_ref, w_ref, o_ref):
    xb = x_ref[...].astype(jnp.bfloat16)
    o_ref[...] = jax.lax.dot_general(
        xb, w_ref[...],
        dimension_numbers=(((1,), (1,)), ((), ())),
        preferred_element_type=jnp.float32)


def kernel(x, W):
    M = x.shape[0]
    Wm = pl.pallas_call(
        _mask_kernel,
        grid=(N_OUT // MASK_BO,),
        in_specs=[pl.BlockSpec((MASK_BO, K_IN), lambda i: (i, 0))],
        out_specs=pl.BlockSpec((MASK_BO, K_IN), lambda i: (i, 0)),
        out_shape=jax.ShapeDtypeStruct((N_OUT, K_IN), jnp.bfloat16),
    )(W)
    out = pl.pallas_call(
        _matmul_kernel,
        grid=(N_OUT // MM_BN, M // MM_BM),
        in_specs=[
            pl.BlockSpec((MM_BM, K_IN), lambda n, m: (m, 0)),
            pl.BlockSpec((MM_BN, K_IN), lambda n, m: (n, 0)),
        ],
        out_specs=pl.BlockSpec((MM_BM, MM_BN), lambda n, m: (m, n)),
        out_shape=jax.ShapeDtypeStruct((M, N_OUT), jnp.float32),
    )(x, Wm)
    return out


# speedup vs baseline: 1.7589x; 1.0003x over previous
import jax
import jax.numpy as jnp
from jax.experimental import pallas as pl
from jax.experimental.pallas import tpu as pltpu

N_OUT = 2048
K_IN = 8192
N_DELAY = 16

MASK_BO = 128
MM_BM = 256
MM_BN = 2048


def _roll_left(x, s):
    return pltpu.roll(x, x.shape[1] - s, 1)


def _mask_kernel(w_ref, o_ref):
    w = w_ref[...]
    a = jnp.abs(w)
    lane_mod = (jax.lax.broadcasted_iota(jnp.int32, (1, K_IN), 1)
                & (N_DELAY - 1))
    neg = jnp.full_like(a, -1.0)
    ok = lane_mod < N_DELAY - 1
    b1 = _roll_left(a, 1)
    t1 = jnp.where(ok, jnp.maximum(a, b1), a)
    t2 = jnp.where(ok, jnp.minimum(a, b1), neg)
    ok = lane_mod < N_DELAY - 2
    b1 = _roll_left(t1, 2)
    b2 = _roll_left(t2, 2)
    m3 = jnp.maximum(jnp.minimum(t1, b2), jnp.minimum(t2, b1))
    m2 = jnp.maximum(jnp.minimum(t1, b1), jnp.maximum(t2, b2))
    t3 = jnp.where(ok, m3, neg)
    m1 = jnp.maximum(t1, b1)
    t2 = jnp.where(ok, m2, t2)
    t1 = jnp.where(ok, m1, t1)
    for s in (4, 8):
        ok = lane_mod < N_DELAY - s
        b1 = _roll_left(t1, s)
        b2 = _roll_left(t2, s)
        b3 = _roll_left(t3, s)
        m3 = jnp.maximum(jnp.maximum(t3, b3),
                         jnp.maximum(jnp.minimum(t1, b2), jnp.minimum(t2, b1)))
        m2 = jnp.maximum(jnp.minimum(t1, b1), jnp.maximum(t2, b2))
        m1 = jnp.maximum(t1, b1)
        t1 = jnp.where(ok, m1, t1)
        t2 = jnp.where(ok, m2, t2)
        t3 = jnp.where(ok, m3, t3)
    for s in (1, 2, 4, 8):
        prv = pltpu.roll(t3, s, 1)
        t3 = jnp.where(lane_mod >= s, prv, t3)
    o_ref[...] = jnp.where(a >= t3, w, 0.0).astype(jnp.bfloat16)


def _matmul_kernel(x_ref, w_ref, o_ref):
    xb = x_ref[...].astype(jnp.bfloat16)
    o_ref[...] = jax.lax.dot_general(
        xb, w_ref[...],
        dimension_numbers=(((1,), (1,)), ((), ())),
        preferred_element_type=jnp.float32)


def kernel(x, W):
    M = x.shape[0]
    Wm = pl.pallas_call(
        _mask_kernel,
        grid=(N_OUT // MASK_BO,),
        in_specs=[pl.BlockSpec((MASK_BO, K_IN), lambda i: (i, 0))],
        out_specs=pl.BlockSpec((MASK_BO, K_IN), lambda i: (i, 0)),
        out_shape=jax.ShapeDtypeStruct((N_OUT, K_IN), jnp.bfloat16),
    )(W)
    out = pl.pallas_call(
        _matmul_kernel,
        grid=(N_OUT // MM_BN, M // MM_BM),
        in_specs=[
            pl.BlockSpec((MM_BM, K_IN), lambda n, m: (m, 0)),
            pl.BlockSpec((MM_BN, K_IN), lambda n, m: (n, 0)),
        ],
        out_specs=pl.BlockSpec((MM_BM, MM_BN), lambda n, m: (m, n)),
        out_shape=jax.ShapeDtypeStruct((M, N_OUT), jnp.float32),
    )(x, Wm)
    return out


# speedup vs baseline: 1.7833x; 1.0139x over previous
import jax
import jax.numpy as jnp
from jax.experimental import pallas as pl
from jax.experimental.pallas import tpu as pltpu

N_OUT = 2048
K_IN = 8192
N_DELAY = 16

MASK_BO = 128
MM_BM = 256
MM_BN = 2048


def _roll_left(x, s):
    return pltpu.roll(x, x.shape[1] - s, 1)


def _mask_kernel(w_ref, o_ref):
    w = w_ref[...]
    a = jnp.abs(w)
    lane_mod = (jax.lax.broadcasted_iota(jnp.int32, (1, K_IN), 1)
                & (N_DELAY - 1))
    neg = jnp.full_like(a, -1.0)
    ok = lane_mod < N_DELAY - 1
    b1 = _roll_left(a, 1)
    t1 = jnp.where(ok, jnp.maximum(a, b1), a)
    t2 = jnp.where(ok, jnp.minimum(a, b1), neg)
    ok = lane_mod < N_DELAY - 2
    b1 = _roll_left(t1, 2)
    b2 = _roll_left(t2, 2)
    m3 = jnp.maximum(jnp.minimum(t1, b2), jnp.minimum(t2, b1))
    m2 = jnp.maximum(jnp.minimum(t1, b1), jnp.maximum(t2, b2))
    t3 = jnp.where(ok, m3, neg)
    m1 = jnp.maximum(t1, b1)
    t2 = jnp.where(ok, m2, t2)
    t1 = jnp.where(ok, m1, t1)
    ok = lane_mod < N_DELAY - 4
    b1 = _roll_left(t1, 4)
    b2 = _roll_left(t2, 4)
    b3 = _roll_left(t3, 4)
    m3 = jnp.maximum(jnp.maximum(t3, b3),
                     jnp.maximum(jnp.minimum(t1, b2), jnp.minimum(t2, b1)))
    m2 = jnp.maximum(jnp.minimum(t1, b1), jnp.maximum(t2, b2))
    m1 = jnp.maximum(t1, b1)
    t1 = jnp.where(ok, m1, t1)
    t2 = jnp.where(ok, m2, t2)
    t3 = jnp.where(ok, m3, t3)
    b1 = _roll_left(t1, 8)
    b2 = _roll_left(t2, 8)
    b3 = _roll_left(t3, 8)
    t3 = jnp.maximum(jnp.maximum(t3, b3),
                     jnp.maximum(jnp.minimum(t1, b2), jnp.minimum(t2, b1)))
    for s in (1, 2, 4, 8):
        prv = pltpu.roll(t3, s, 1)
        t3 = jnp.where(lane_mod >= s, prv, t3)
    o_ref[...] = jnp.where(a >= t3, w, 0.0).astype(jnp.bfloat16)


def _matmul_kernel(x_ref, w_ref, o_ref):
    xb = x_ref[...].astype(jnp.bfloat16)
    o_ref[...] = jax.lax.dot_general(
        xb, w_ref[...],
        dimension_numbers=(((1,), (1,)), ((), ())),
        preferred_element_type=jnp.float32)


def kernel(x, W):
    M = x.shape[0]
    Wm = pl.pallas_call(
        _mask_kernel,
        grid=(N_OUT // MASK_BO,),
        in_specs=[pl.BlockSpec((MASK_BO, K_IN), lambda i: (i, 0))],
        out_specs=pl.BlockSpec((MASK_BO, K_IN), lambda i: (i, 0)),
        out_shape=jax.ShapeDtypeStruct((N_OUT, K_IN), jnp.bfloat16),
    )(W)
    out = pl.pallas_call(
        _matmul_kernel,
        grid=(N_OUT // MM_BN, M // MM_BM),
        in_specs=[
            pl.BlockSpec((MM_BM, K_IN), lambda n, m: (m, 0)),
            pl.BlockSpec((MM_BN, K_IN), lambda n, m: (n, 0)),
        ],
        out_specs=pl.BlockSpec((MM_BM, MM_BN), lambda n, m: (m, n)),
        out_shape=jax.ShapeDtypeStruct((M, N_OUT), jnp.float32),
    )(x, Wm)
    return out


# speedup vs baseline: 1.7949x; 1.0065x over previous
import jax
import jax.numpy as jnp
from jax.experimental import pallas as pl
from jax.experimental.pallas import tpu as pltpu

N_OUT = 2048
K_IN = 8192
N_DELAY = 16

MASK_BO = 128
MM_BM = 256
MM_BN = 2048


def _roll_left(x, s):
    return pltpu.roll(x, x.shape[1] - s, 1)


def _mask_kernel(w_ref, o_ref):
    w = w_ref[...]
    a = jnp.abs(w)
    lane_mod = (jax.lax.broadcasted_iota(jnp.int32, (1, K_IN), 1)
                & (N_DELAY - 1))
    b1 = _roll_left(a, 1)
    t1 = jnp.maximum(a, b1)
    t2 = jnp.minimum(a, b1)
    b1 = _roll_left(t1, 2)
    b2 = _roll_left(t2, 2)
    t3 = jnp.maximum(jnp.minimum(t1, b2), jnp.minimum(t2, b1))
    m2 = jnp.maximum(jnp.minimum(t1, b1), jnp.maximum(t2, b2))
    t1 = jnp.maximum(t1, b1)
    t2 = m2
    b1 = _roll_left(t1, 4)
    b2 = _roll_left(t2, 4)
    b3 = _roll_left(t3, 4)
    m3 = jnp.maximum(jnp.maximum(t3, b3),
                     jnp.maximum(jnp.minimum(t1, b2), jnp.minimum(t2, b1)))
    m2 = jnp.maximum(jnp.minimum(t1, b1), jnp.maximum(t2, b2))
    t1 = jnp.maximum(t1, b1)
    t2 = m2
    t3 = m3
    b1 = _roll_left(t1, 8)
    b2 = _roll_left(t2, 8)
    b3 = _roll_left(t3, 8)
    t3 = jnp.maximum(jnp.maximum(t3, b3),
                     jnp.maximum(jnp.minimum(t1, b2), jnp.minimum(t2, b1)))
    for s in (1, 2, 4, 8):
        prv = pltpu.roll(t3, s, 1)
        t3 = jnp.where(lane_mod >= s, prv, t3)
    o_ref[...] = jnp.where(a >= t3, w, 0.0).astype(jnp.bfloat16)


def _matmul_kernel(x_ref, w_ref, o_ref):
    xb = x_ref[...].astype(jnp.bfloat16)
    o_ref[...] = jax.lax.dot_general(
        xb, w_ref[...],
        dimension_numbers=(((1,), (1,)), ((), ())),
        preferred_element_type=jnp.float32)


def kernel(x, W):
    M = x.shape[0]
    Wm = pl.pallas_call(
        _mask_kernel,
        grid=(N_OUT // MASK_BO,),
        in_specs=[pl.BlockSpec((MASK_BO, K_IN), lambda i: (i, 0))],
        out_specs=pl.BlockSpec((MASK_BO, K_IN), lambda i: (i, 0)),
        out_shape=jax.ShapeDtypeStruct((N_OUT, K_IN), jnp.bfloat16),
    )(W)
    out = pl.pallas_call(
        _matmul_kernel,
        grid=(N_OUT // MM_BN, M // MM_BM),
        in_specs=[
            pl.BlockSpec((MM_BM, K_IN), lambda n, m: (m, 0)),
            pl.BlockSpec((MM_BN, K_IN), lambda n, m: (n, 0)),
        ],
        out_specs=pl.BlockSpec((MM_BM, MM_BN), lambda n, m: (m, n)),
        out_shape=jax.ShapeDtypeStruct((M, N_OUT), jnp.float32),
    )(x, Wm)
    return out
